# baseline (speedup 1.0000x reference)
import functools

import jax
import jax.numpy as jnp
from jax import lax
from jax.experimental import pallas as pl
from jax.experimental.pallas import tpu as pltpu
from jax.experimental.pallas import tpu_sc as plsc

N = 10000
E = 160000
D = 256
H = 128
C = 64

NC = 2
NS = 16
NW = NC * NS
CH = 128
E_PAD = 163840
NCHUNK_TOTAL = E_PAD // CH
CPW = NCHUNK_TOTAL // NW
N_PAD = 10240
RPS = N_PAD // NS


def _sc_mesh():
    return plsc.VectorSubcoreMesh(
        core_axis_name="c", subcore_axis_name="s", num_cores=NC, num_subcores=NS
    )


def _hist_body(dst_i, ones_hbm, zeros_hbm, out, idx_d, ones_v, acc, sem):
    c = lax.axis_index("c")
    s = lax.axis_index("s")
    wid = c * NS + s
    pltpu.sync_copy(
        zeros_hbm.at[pl.ds(s * RPS, RPS)], acc.at[pl.ds(s * RPS, RPS)]
    )
    pltpu.sync_copy(dst_i.at[pl.ds(wid * CPW, CPW)], idx_d)
    pltpu.sync_copy(ones_hbm, ones_v)
    plsc.subcore_barrier()

    def chunk(j, carry):
        pltpu.async_copy(ones_v, acc.at[idx_d.at[j]], sem, add=True)
        return carry

    lax.fori_loop(0, CPW, chunk, 0)

    def drain(j, carry):
        pltpu.make_async_copy(ones_v, acc.at[idx_d.at[j]], sem).wait()
        return carry

    lax.fori_loop(0, CPW, drain, 0)
    plsc.subcore_barrier()
    pltpu.sync_copy(
        acc.at[pl.ds(s * RPS, RPS)], out.at[c, pl.ds(s * RPS, RPS)]
    )


def _make_hist():
    return pl.kernel(
        _hist_body,
        out_type=jax.ShapeDtypeStruct((NC, N_PAD, H), jnp.float32),
        mesh=_sc_mesh(),
        scratch_types=[
            pltpu.VMEM((CPW, CH), jnp.int32),
            pltpu.VMEM((CH, H), jnp.float32),
            pltpu.VMEM_SHARED((N_PAD, H), jnp.float32),
            pltpu.SemaphoreType.DMA,
        ],
    )


NBUF = 2


def _prop_body(F, g_hbm, src_i, dst_i, zeros_hbm, out, idx_s, idx_d, rows, acc, *sems):
    gsems = sems[:NBUF]
    ssems = sems[NBUF:]
    c = lax.axis_index("c")
    s = lax.axis_index("s")
    wid = c * NS + s
    pltpu.sync_copy(
        zeros_hbm.at[pl.ds(s * RPS, RPS)], acc.at[pl.ds(s * RPS, RPS)]
    )
    pltpu.sync_copy(src_i.at[pl.ds(wid * CPW, CPW)], idx_s)
    pltpu.sync_copy(dst_i.at[pl.ds(wid * CPW, CPW)], idx_d)
    plsc.subcore_barrier()

    pltpu.async_copy(g_hbm.at[idx_s.at[0]], rows.at[0], gsems[0])

    def pair(j2, carry):
        j0 = NBUF * j2
        for b in range(NBUF):
            j = j0 + b
            nb = 1 - b
            pltpu.make_async_copy(g_hbm.at[idx_s.at[j]], rows.at[b], gsems[b]).wait()

            @pl.when(j >= 1)
            def _():
                pltpu.make_async_copy(
                    rows.at[nb], acc.at[idx_d.at[j - 1]], ssems[nb]
                ).wait()

            pltpu.async_copy(rows.at[b], acc.at[idx_d.at[j]], ssems[b], add=True)

            @pl.when(j + 1 < CPW)
            def _():
                pltpu.async_copy(g_hbm.at[idx_s.at[j + 1]], rows.at[nb], gsems[nb])
        return carry

    lax.fori_loop(0, CPW // NBUF, pair, 0)
    b = (CPW - 1) % NBUF
    pltpu.make_async_copy(rows.at[b], acc.at[idx_d.at[CPW - 1]], ssems[b]).wait()
    plsc.subcore_barrier()
    pltpu.sync_copy(
        acc.at[pl.ds(s * RPS, RPS)], out.at[c, pl.ds(s * RPS, RPS)]
    )


def _make_prop(F):
    return pl.kernel(
        functools.partial(_prop_body, F),
        out_type=jax.ShapeDtypeStruct((NC, N_PAD, F), jnp.float32),
        mesh=_sc_mesh(),
        scratch_types=[
            pltpu.VMEM((CPW, CH), jnp.int32),
            pltpu.VMEM((CPW, CH), jnp.int32),
            pltpu.VMEM((NBUF, CH, F), jnp.float32),
            pltpu.VMEM_SHARED((N_PAD, F), jnp.float32),
        ]
        + [pltpu.SemaphoreType.DMA] * (2 * NBUF),
    )


BR = 400


def _mm1_body(x_ref, w1_ref, o_ref):
    o_ref[...] = jnp.dot(x_ref[...], w1_ref[...], preferred_element_type=jnp.float32)


def _mm1(x, W1):
    return pl.pallas_call(
        _mm1_body,
        grid=(N // BR,),
        in_specs=[
            pl.BlockSpec((BR, D), lambda i: (i, 0)),
            pl.BlockSpec((D, H), lambda i: (0, 0)),
        ],
        out_specs=pl.BlockSpec((BR, H), lambda i: (i, 0)),
        out_shape=jax.ShapeDtypeStruct((N, H), jnp.float32),
    )(x, W1)


def _dinv_from(degp_ref):
    deg = degp_ref[0, :, 0] + degp_ref[1, :, 0] + 1.0
    return lax.rsqrt(deg)


def _scale_body(xw_ref, degp_ref, o_ref):
    dinv = _dinv_from(degp_ref)
    o_ref[...] = xw_ref[...] * dinv[:, None]


def _scale(xw, degp):
    return pl.pallas_call(
        _scale_body,
        grid=(N // BR,),
        in_specs=[
            pl.BlockSpec((BR, H), lambda i: (i, 0)),
            pl.BlockSpec((NC, BR, H), lambda i: (0, i, 0)),
        ],
        out_specs=pl.BlockSpec((BR, H), lambda i: (i, 0)),
        out_shape=jax.ShapeDtypeStruct((N, H), jnp.float32),
    )(xw, degp)


def _mm2_body(s1_ref, p_ref, degp_ref, b1_ref, w2_ref, o_ref):
    dinv = _dinv_from(degp_ref)
    h1 = (s1_ref[0] + s1_ref[1] + p_ref[...]) * dinv[:, None] + b1_ref[...]
    h1 = jnp.maximum(h1, 0.0)
    q = jnp.dot(h1, w2_ref[...], preferred_element_type=jnp.float32)
    o_ref[...] = q * dinv[:, None]


def _mm2(s1, p, degp, b1, W2p):
    return pl.pallas_call(
        _mm2_body,
        grid=(N // BR,),
        in_specs=[
            pl.BlockSpec((NC, BR, H), lambda i: (0, i, 0)),
            pl.BlockSpec((BR, H), lambda i: (i, 0)),
            pl.BlockSpec((NC, BR, H), lambda i: (0, i, 0)),
            pl.BlockSpec((1, H), lambda i: (0, 0)),
            pl.BlockSpec((H, H), lambda i: (0, 0)),
        ],
        out_specs=pl.BlockSpec((BR, H), lambda i: (i, 0)),
        out_shape=jax.ShapeDtypeStruct((N, H), jnp.float32),
    )(s1, p, degp, b1, W2p)


def _final_body(s2_ref, q_ref, degp_ref, b2_ref, o_ref):
    dinv = _dinv_from(degp_ref)
    full = (s2_ref[0] + s2_ref[1] + q_ref[...]) * dinv[:, None]
    o_ref[...] = full[:, :C] + b2_ref[...]


def _final(s2, q, degp, b2):
    return pl.pallas_call(
        _final_body,
        grid=(N // BR,),
        in_specs=[
            pl.BlockSpec((NC, BR, H), lambda i: (0, i, 0)),
            pl.BlockSpec((BR, H), lambda i: (i, 0)),
            pl.BlockSpec((NC, BR, H), lambda i: (0, i, 0)),
            pl.BlockSpec((1, C), lambda i: (0, 0)),
        ],
        out_specs=pl.BlockSpec((BR, C), lambda i: (i, 0)),
        out_shape=jax.ShapeDtypeStruct((N, C), jnp.float32),
    )(s2, q, degp, b2)


@jax.jit
def kernel(x, edge_index, W1, b1, W2, b2):
    src = edge_index[0]
    dst = edge_index[1]
    pad = E_PAD - E
    src_p = jnp.concatenate([src, jnp.zeros((pad,), jnp.int32)]).reshape(
        NCHUNK_TOTAL, CH
    )
    dst_p = jnp.concatenate([dst, jnp.full((pad,), N, jnp.int32)]).reshape(
        NCHUNK_TOTAL, CH
    )

    onesH = jnp.ones((CH, H), jnp.float32)
    zerosH = jnp.zeros((N_PAD, H), jnp.float32)
    W2p = jnp.pad(W2, ((0, 0), (0, H - C)))

    degp = _make_hist()(dst_p, onesH, zerosH)
    xw1 = _mm1(x, W1)
    p = _scale(xw1, degp)
    s1 = _make_prop(H)(p, src_p, dst_p, zerosH)
    q = _mm2(s1, p, degp, b1.reshape(1, H), W2p)
    s2 = _make_prop(H)(q, src_p, dst_p, zerosH)
    return _final(s2, q, degp, b2.reshape(1, C))


# speedup vs baseline: 1.1533x; 1.1533x over previous
import functools

import jax
import jax.numpy as jnp
from jax import lax
from jax.experimental import pallas as pl
from jax.experimental.pallas import tpu as pltpu
from jax.experimental.pallas import tpu_sc as plsc

N = 10000
E = 160000
D = 256
H = 128
C = 64

NC = 2
NS = 16
NW = NC * NS
CH = 128
E_PAD = 163840
NCHUNK_TOTAL = E_PAD // CH
CPW = NCHUNK_TOTAL // NW
N_PAD = 10240
RPS = N_PAD // NS


def _sc_mesh():
    return plsc.VectorSubcoreMesh(
        core_axis_name="c", subcore_axis_name="s", num_cores=NC, num_subcores=NS
    )


def _hist_body(dst_i, ones_hbm, zeros_hbm, out, idx_d, ones_v, acc, sem):
    c = lax.axis_index("c")
    s = lax.axis_index("s")
    wid = c * NS + s
    pltpu.sync_copy(
        zeros_hbm.at[pl.ds(s * RPS, RPS)], acc.at[pl.ds(s * RPS, RPS)]
    )
    pltpu.sync_copy(dst_i.at[pl.ds(wid * CPW, CPW)], idx_d)
    pltpu.sync_copy(ones_hbm, ones_v)
    plsc.subcore_barrier()

    def chunk(j, carry):
        pltpu.async_copy(ones_v, acc.at[idx_d.at[j]], sem, add=True)
        return carry

    lax.fori_loop(0, CPW, chunk, 0)

    def drain(j, carry):
        pltpu.make_async_copy(ones_v, acc.at[idx_d.at[j]], sem).wait()
        return carry

    lax.fori_loop(0, CPW, drain, 0)
    plsc.subcore_barrier()
    pltpu.sync_copy(
        acc.at[pl.ds(s * RPS, RPS)], out.at[c, pl.ds(s * RPS, RPS)]
    )


def _make_hist():
    return pl.kernel(
        _hist_body,
        out_type=jax.ShapeDtypeStruct((NC, N_PAD, H), jnp.float32),
        mesh=_sc_mesh(),
        scratch_types=[
            pltpu.VMEM((CPW, CH), jnp.int32),
            pltpu.VMEM((CH, H), jnp.float32),
            pltpu.VMEM_SHARED((N_PAD, H), jnp.float32),
            pltpu.SemaphoreType.DMA,
        ],
    )


NBUF = 2


def _prop_body(F, g_hbm, src_i, dst_i, zeros_hbm, out, idx_s, idx_d, rows, acc, *sems):
    gsems = sems[:NBUF]
    ssems = sems[NBUF:]
    c = lax.axis_index("c")
    s = lax.axis_index("s")
    wid = c * NS + s
    pltpu.sync_copy(
        zeros_hbm.at[pl.ds(s * RPS, RPS)], acc.at[pl.ds(s * RPS, RPS)]
    )
    pltpu.sync_copy(src_i.at[pl.ds(wid * CPW, CPW)], idx_s)
    pltpu.sync_copy(dst_i.at[pl.ds(wid * CPW, CPW)], idx_d)
    plsc.subcore_barrier()

    pltpu.async_copy(g_hbm.at[idx_s.at[0]], rows.at[0], gsems[0])

    def pair(j2, carry):
        j0 = NBUF * j2
        for b in range(NBUF):
            j = j0 + b
            nb = 1 - b
            pltpu.make_async_copy(g_hbm.at[idx_s.at[j]], rows.at[b], gsems[b]).wait()

            @pl.when(j >= 1)
            def _():
                pltpu.make_async_copy(
                    rows.at[nb], acc.at[idx_d.at[j - 1]], ssems[nb]
                ).wait()

            pltpu.async_copy(rows.at[b], acc.at[idx_d.at[j]], ssems[b], add=True)

            @pl.when(j + 1 < CPW)
            def _():
                pltpu.async_copy(g_hbm.at[idx_s.at[j + 1]], rows.at[nb], gsems[nb])
        return carry

    lax.fori_loop(0, CPW // NBUF, pair, 0)
    b = (CPW - 1) % NBUF
    pltpu.make_async_copy(rows.at[b], acc.at[idx_d.at[CPW - 1]], ssems[b]).wait()
    plsc.subcore_barrier()
    pltpu.sync_copy(
        acc.at[pl.ds(s * RPS, RPS)], out.at[c, pl.ds(s * RPS, RPS)]
    )


def _make_prop(F):
    return pl.kernel(
        functools.partial(_prop_body, F),
        out_type=jax.ShapeDtypeStruct((NC, N_PAD, F), jnp.float32),
        mesh=_sc_mesh(),
        scratch_types=[
            pltpu.VMEM((CPW, CH), jnp.int32),
            pltpu.VMEM((CPW, CH), jnp.int32),
            pltpu.VMEM((NBUF, CH, F), jnp.float32),
            pltpu.VMEM_SHARED((N_PAD, F), jnp.float32),
        ]
        + [pltpu.SemaphoreType.DMA] * (2 * NBUF),
    )


BR = 400


def _mm1_body(x_ref, w1_ref, o_ref):
    o_ref[...] = jnp.dot(x_ref[...], w1_ref[...], preferred_element_type=jnp.float32)


def _mm1(x, W1):
    return pl.pallas_call(
        _mm1_body,
        grid=(N // BR,),
        in_specs=[
            pl.BlockSpec((BR, D), lambda i: (i, 0)),
            pl.BlockSpec((D, H), lambda i: (0, 0)),
        ],
        out_specs=pl.BlockSpec((BR, H), lambda i: (i, 0)),
        out_shape=jax.ShapeDtypeStruct((N, H), jnp.float32),
    )(x, W1)


def _dinv_from(degp_ref):
    deg = degp_ref[0, :, 0] + degp_ref[1, :, 0] + 1.0
    return lax.rsqrt(deg)


def _scale_body(xw_ref, degp_ref, o_ref):
    dinv = _dinv_from(degp_ref)
    o_ref[...] = xw_ref[...] * dinv[:, None]


def _scale(xw, degp):
    return pl.pallas_call(
        _scale_body,
        grid=(N // BR,),
        in_specs=[
            pl.BlockSpec((BR, H), lambda i: (i, 0)),
            pl.BlockSpec((NC, BR, H), lambda i: (0, i, 0)),
        ],
        out_specs=pl.BlockSpec((BR, H), lambda i: (i, 0)),
        out_shape=jax.ShapeDtypeStruct((N, H), jnp.float32),
    )(xw, degp)


def _mm2_body(s1_ref, p_ref, degp_ref, b1_ref, w2_ref, o_ref):
    dinv = _dinv_from(degp_ref)
    h1 = (s1_ref[0] + s1_ref[1] + p_ref[...]) * dinv[:, None] + b1_ref[...]
    h1 = jnp.maximum(h1, 0.0)
    q = jnp.dot(h1, w2_ref[...], preferred_element_type=jnp.float32)
    o_ref[...] = q * dinv[:, None]


def _mm2(s1, p, degp, b1, W2p):
    return pl.pallas_call(
        _mm2_body,
        grid=(N // BR,),
        in_specs=[
            pl.BlockSpec((NC, BR, H), lambda i: (0, i, 0)),
            pl.BlockSpec((BR, H), lambda i: (i, 0)),
            pl.BlockSpec((NC, BR, H), lambda i: (0, i, 0)),
            pl.BlockSpec((1, H), lambda i: (0, 0)),
            pl.BlockSpec((H, H), lambda i: (0, 0)),
        ],
        out_specs=pl.BlockSpec((BR, H), lambda i: (i, 0)),
        out_shape=jax.ShapeDtypeStruct((N, H), jnp.float32),
    )(s1, p, degp, b1, W2p)


def _final_body(s2_ref, q_ref, degp_ref, b2_ref, o_ref):
    dinv = _dinv_from(degp_ref)
    full = (s2_ref[0] + s2_ref[1] + q_ref[...]) * dinv[:, None]
    o_ref[...] = full[:, :C] + b2_ref[...]


def _final(s2, q, degp, b2):
    return pl.pallas_call(
        _final_body,
        grid=(N // BR,),
        in_specs=[
            pl.BlockSpec((NC, BR, H), lambda i: (0, i, 0)),
            pl.BlockSpec((BR, H), lambda i: (i, 0)),
            pl.BlockSpec((NC, BR, H), lambda i: (0, i, 0)),
            pl.BlockSpec((1, C), lambda i: (0, 0)),
        ],
        out_specs=pl.BlockSpec((BR, C), lambda i: (i, 0)),
        out_shape=jax.ShapeDtypeStruct((N, C), jnp.float32),
    )(s2, q, degp, b2)


@jax.jit
def kernel(x, edge_index, W1, b1, W2, b2):
    src = edge_index[0]
    dst = edge_index[1]
    pad = E_PAD - E
    pad_dst = N + (jnp.arange(pad, dtype=jnp.int32) % (N_PAD - N))

    def chunked(idx):
        return idx.reshape(CPW, NW, CH).transpose(1, 0, 2).reshape(NCHUNK_TOTAL, CH)

    src_p = chunked(jnp.concatenate([src, jnp.zeros((pad,), jnp.int32)]))
    dst_p = chunked(jnp.concatenate([dst, pad_dst]))

    onesH = jnp.ones((CH, H), jnp.float32)
    zerosH = jnp.zeros((N_PAD, H), jnp.float32)
    W2p = jnp.pad(W2, ((0, 0), (0, H - C)))

    degp = _make_hist()(dst_p, onesH, zerosH)
    xw1 = _mm1(x, W1)
    p = _scale(xw1, degp)
    s1 = _make_prop(H)(p, src_p, dst_p, zerosH)
    q = _mm2(s1, p, degp, b1.reshape(1, H), W2p)
    s2 = _make_prop(H)(q, src_p, dst_p, zerosH)
    return _final(s2, q, degp, b2.reshape(1, C))
